# baseline (speedup 1.0000x reference)
import jax
import jax.numpy as jnp
from jax import lax
from jax.experimental import pallas as pl
from jax.experimental.pallas import tpu as pltpu

B, S, D = 4, 8192, 1024
CHUNK = 512
N_POS = S // CHUNK
N_CH = B * S // CHUNK
R_IN = 4
R_OUT = 4


def _body(x_hbm, p_hbm, o_hbm, xv, pv, ov, sin, spos, sout):
    for k in range(N_POS):
        pltpu.make_async_copy(
            p_hbm.at[pl.ds(k * CHUNK, CHUNK)], pv.at[k], spos.at[k]).start()
    for r in range(R_IN):
        pltpu.make_async_copy(
            x_hbm.at[pl.ds(r * CHUNK, CHUNK)], xv.at[r], sin.at[r]).start()

    def step(c, carry):
        r = lax.rem(c, R_IN)
        ro = lax.rem(c, R_OUT)
        sc = lax.rem(c, N_POS)
        pltpu.make_async_copy(
            x_hbm.at[pl.ds(c * CHUNK, CHUNK)], xv.at[r], sin.at[r]).wait()

        @pl.when(c < N_POS)
        def _():
            pltpu.make_async_copy(
                p_hbm.at[pl.ds(sc * CHUNK, CHUNK)], pv.at[sc],
                spos.at[sc]).wait()

        @pl.when(c >= R_OUT)
        def _():
            pltpu.make_async_copy(
                ov.at[ro], o_hbm.at[pl.ds((c - R_OUT) * CHUNK, CHUNK)],
                sout.at[ro]).wait()

        ov[ro] = xv[r] + pv[sc]
        pltpu.make_async_copy(
            ov.at[ro], o_hbm.at[pl.ds(c * CHUNK, CHUNK)], sout.at[ro]).start()

        @pl.when(c + R_IN < N_CH)
        def _():
            pltpu.make_async_copy(
                x_hbm.at[pl.ds((c + R_IN) * CHUNK, CHUNK)], xv.at[r],
                sin.at[r]).start()

        return carry

    lax.fori_loop(0, N_CH, step, 0)
    for k in range(R_OUT):
        c = N_CH - R_OUT + k
        pltpu.make_async_copy(
            ov.at[c % R_OUT], o_hbm.at[pl.ds(c * CHUNK, CHUNK)],
            sout.at[c % R_OUT]).wait()


def kernel(x, pos_embed):
    x2 = x.reshape(B * S, D)
    out = pl.pallas_call(
        _body,
        in_specs=[
            pl.BlockSpec(memory_space=pltpu.MemorySpace.HBM),
            pl.BlockSpec(memory_space=pltpu.MemorySpace.HBM),
        ],
        out_specs=pl.BlockSpec(memory_space=pltpu.MemorySpace.HBM),
        out_shape=jax.ShapeDtypeStruct((B * S, D), x.dtype),
        scratch_shapes=[
            pltpu.VMEM((R_IN, CHUNK, D), jnp.float32),
            pltpu.VMEM((N_POS, CHUNK, D), jnp.float32),
            pltpu.VMEM((R_OUT, CHUNK, D), jnp.float32),
            pltpu.SemaphoreType.DMA((R_IN,)),
            pltpu.SemaphoreType.DMA((N_POS,)),
            pltpu.SemaphoreType.DMA((R_OUT,)),
        ],
        compiler_params=pltpu.CompilerParams(vmem_limit_bytes=60 * 1024 * 1024),
    )(x2, pos_embed)
    return out.reshape(B, S, D)
